# baseline (speedup 1.0000x reference)
import functools
import math

import jax
import jax.numpy as jnp
from jax.experimental import pallas as pl
from jax.experimental.pallas import tpu as pltpu

_NEG = -3.0e38


def _attn_block(t, wc, bc, wq, wk, wv, wo, bo, *, heads, topk, exact_agg):
    bf = lambda a: a.astype(jnp.bfloat16)
    dot = lambda a, b: jnp.dot(bf(a), bf(b), preferred_element_type=jnp.float32)
    conv = dot(t, wc) + bc
    q = dot(t, wq)
    k = dot(t, wk)
    v = dot(t, wv)
    n, d = q.shape
    dh = d // heads
    scale = jnp.sqrt(jnp.float32(dh))
    col = jax.lax.broadcasted_iota(jnp.int32, (n, d), 1)

    kb = bf(k)
    sims = []
    for h in range(heads):
        qm = jnp.where((col >= h * dh) & (col < (h + 1) * dh), q, 0.0)
        sims.append(jax.lax.dot_general(
            bf(qm), kb, (((1,), (1,)), ((), ())),
            preferred_element_type=jnp.float32) / scale)
    sim = jnp.concatenate(sims, axis=0)

    r = heads * n
    lanef = jax.lax.broadcasted_iota(jnp.int32, (r, n), 1).astype(jnp.float32)
    cur = sim
    rmax = None
    for i in range(topk):
        m = jnp.max(cur, axis=-1, keepdims=True)
        if i == 0:
            rmax = m
        cand = jnp.where(cur == m, lanef, 1e9)
        imin = jnp.min(cand, axis=-1, keepdims=True)
        cur = jnp.where(cand == imin, _NEG, cur)
    e = jnp.where(cur < -1.0e37, jnp.exp(sim - rmax), 0.0)
    attn = e / jnp.sum(e, axis=-1, keepdims=True)

    agg = jnp.zeros((n, d), dtype=jnp.float32)
    for h in range(heads):
        vm = jnp.where((col >= h * dh) & (col < (h + 1) * dh), v, 0.0)
        ah = attn[h * n:(h + 1) * n]
        if exact_agg:
            agg = agg + jnp.dot(ah, vm, precision=jax.lax.Precision.HIGHEST,
                                preferred_element_type=jnp.float32)
        else:
            agg = agg + dot(ah, vm)

    cat = jnp.concatenate([conv, agg], axis=-1)
    out = dot(cat, wo) + bo
    return jnp.maximum(out, 0.0)


def _layer_body(t_ref, wc_ref, bc_ref, wq_ref, wk_ref, wv_ref, wo_ref, bo_ref,
                o_ref, *, heads, topk, exact_agg, imgs):
    for j in range(imgs):
        o_ref[j] = _attn_block(
            t_ref[j], wc_ref[...], bc_ref[...], wq_ref[...], wk_ref[...],
            wv_ref[...], wo_ref[...], bo_ref[...],
            heads=heads, topk=topk, exact_agg=exact_agg)


def _attn_layer(t, ws, *, heads, topk, exact_agg, imgs=2):
    b, n, c = t.shape
    co = ws[-2].shape[1]
    ws = [w.reshape(1, -1) if w.ndim == 1 else w for w in ws]
    full = lambda w: pl.BlockSpec(w.shape, lambda i: (0,) * w.ndim)
    return pl.pallas_call(
        functools.partial(_layer_body, heads=heads, topk=topk,
                          exact_agg=exact_agg, imgs=imgs),
        grid=(b // imgs,),
        in_specs=[pl.BlockSpec((imgs, n, c), lambda i: (i, 0, 0))]
                 + [full(w) for w in ws],
        out_specs=pl.BlockSpec((imgs, n, co), lambda i: (i, 0, 0)),
        out_shape=jax.ShapeDtypeStruct((b, n, co), jnp.float32),
        compiler_params=pltpu.CompilerParams(
            dimension_semantics=("arbitrary",)),
    )(t, *ws)


def _fc_body(f_ref, w1_ref, b1_ref, w2_ref, b2_ref, o_ref, acc_ref, *, nk):
    ki = pl.program_id(0)

    @pl.when(ki == 0)
    def _init():
        acc_ref[...] = jnp.zeros_like(acc_ref)

    fb = f_ref[...].astype(jnp.bfloat16)
    wb = w1_ref[...].astype(jnp.bfloat16)
    acc_ref[...] += jnp.dot(fb, wb, preferred_element_type=jnp.float32)

    @pl.when(ki == nk - 1)
    def _fin():
        h = jnp.maximum(acc_ref[...] + b1_ref[...], 0.0)
        o_ref[...] = jnp.dot(h.astype(jnp.bfloat16),
                             w2_ref[...].astype(jnp.bfloat16),
                             preferred_element_type=jnp.float32) + b2_ref[...]


def _classifier(f, w1, b1, w2, b2, *, kblk=4096):
    b, ktot = f.shape
    hid = w1.shape[1]
    ncls = w2.shape[1]
    nk = ktot // kblk
    b1 = b1.reshape(1, -1)
    b2 = b2.reshape(1, -1)
    return pl.pallas_call(
        functools.partial(_fc_body, nk=nk),
        grid=(nk,),
        in_specs=[
            pl.BlockSpec((b, kblk), lambda i: (0, i)),
            pl.BlockSpec((kblk, hid), lambda i: (i, 0)),
            pl.BlockSpec((1, hid), lambda i: (0, 0)),
            pl.BlockSpec((hid, ncls), lambda i: (0, 0)),
            pl.BlockSpec((1, ncls), lambda i: (0, 0)),
        ],
        out_specs=pl.BlockSpec((b, ncls), lambda i: (0, 0)),
        out_shape=jax.ShapeDtypeStruct((b, ncls), jnp.float32),
        scratch_shapes=[pltpu.VMEM((b, hid), jnp.float32)],
        compiler_params=pltpu.CompilerParams(
            dimension_semantics=("arbitrary",)),
    )(f, w1, b1, w2, b2)


def _unshuffle_tokens(x, r):
    b, c, hh, ww = x.shape
    x = x.reshape(b, c, hh // r, r, ww // r, r)
    x = x.transpose(0, 1, 3, 5, 2, 4)
    x = x.reshape(b, c * r * r, (hh // r) * (ww // r))
    return x.transpose(0, 2, 1)


def kernel(x, Wc1, bc1, Wq1, Wk1, Wv1, Wo1, bo1, Wc2, bc2, Wq2, Wk2, Wv2, Wo2,
           bo2, W1, b1, W2, b2):
    t1 = _unshuffle_tokens(x, 2)
    h1 = _attn_layer(t1, [Wc1, bc1, Wq1, Wk1, Wv1, Wo1, bo1],
                     heads=4, topk=9, exact_agg=True)
    h2 = _attn_layer(h1, [Wc2, bc2, Wq2, Wk2, Wv2, Wo2, bo2],
                     heads=4, topk=9, exact_agg=False)
    b, n, co = h2.shape
    hs = int(math.isqrt(n))
    g = h2.transpose(0, 2, 1).reshape(b, co, hs, hs)
    r = 2
    g = g.reshape(b, co // (r * r), r, r, hs, hs)
    g = g.transpose(0, 1, 4, 2, 5, 3).reshape(b, co // (r * r), hs * r, hs * r)
    f = g.reshape(b, -1)
    return _classifier(f, W1, b1, W2, b2)


# speedup vs baseline: 1.0386x; 1.0386x over previous
import functools
import math

import jax
import jax.numpy as jnp
from jax.experimental import pallas as pl
from jax.experimental.pallas import tpu as pltpu

_NEG = -3.0e38


def _attn_block(t, wc, bc, wq, wk, wv, wo, bo, *, heads, topk, exact_agg):
    bf = lambda a: a.astype(jnp.bfloat16)
    dot = lambda a, b: jnp.dot(bf(a), bf(b), preferred_element_type=jnp.float32)
    conv = dot(t, wc) + bc
    q = dot(t, wq)
    k = dot(t, wk)
    v = dot(t, wv)
    n, d = q.shape
    dh = d // heads
    scale = jnp.sqrt(jnp.float32(dh))
    col = jax.lax.broadcasted_iota(jnp.int32, (n, d), 1)

    kb = bf(k)
    sims = []
    for h in range(heads):
        qm = jnp.where((col >= h * dh) & (col < (h + 1) * dh), q, 0.0)
        sims.append(jax.lax.dot_general(
            bf(qm), kb, (((1,), (1,)), ((), ())),
            preferred_element_type=jnp.float32) / scale)
    sim = jnp.concatenate(sims, axis=0)

    r = heads * n
    lanef = jax.lax.broadcasted_iota(jnp.int32, (r, n), 1).astype(jnp.float32)
    cur = sim
    rmax = None
    for i in range(topk):
        m = jnp.max(cur, axis=-1, keepdims=True)
        if i == 0:
            rmax = m
        cand = jnp.where(cur == m, lanef, 1e9)
        imin = jnp.min(cand, axis=-1, keepdims=True)
        cur = jnp.where(cand == imin, _NEG, cur)
    e = jnp.where(cur < -1.0e37, jnp.exp(sim - rmax), 0.0)
    attn = e / jnp.sum(e, axis=-1, keepdims=True)

    agg = jnp.zeros((n, d), dtype=jnp.float32)
    for h in range(heads):
        vm = jnp.where((col >= h * dh) & (col < (h + 1) * dh), v, 0.0)
        ah = attn[h * n:(h + 1) * n]
        if exact_agg:
            agg = agg + jnp.dot(ah, vm, precision=jax.lax.Precision.HIGHEST,
                                preferred_element_type=jnp.float32)
        else:
            agg = agg + dot(ah, vm)

    cat = jnp.concatenate([conv, agg], axis=-1)
    out = dot(cat, wo) + bo
    return jnp.maximum(out, 0.0)


def _layer_body(t_ref, wc_ref, bc_ref, wq_ref, wk_ref, wv_ref, wo_ref, bo_ref,
                o_ref, *, heads, topk, exact_agg, imgs):
    for j in range(imgs):
        o_ref[j] = _attn_block(
            t_ref[j], wc_ref[...], bc_ref[...], wq_ref[...], wk_ref[...],
            wv_ref[...], wo_ref[...], bo_ref[...],
            heads=heads, topk=topk, exact_agg=exact_agg)


def _attn_layer(t, ws, *, heads, topk, exact_agg, imgs=1):
    b, n, c = t.shape
    co = ws[-2].shape[1]
    ws = [w.reshape(1, -1) if w.ndim == 1 else w for w in ws]
    full = lambda w: pl.BlockSpec(w.shape, lambda i: (0,) * w.ndim)
    return pl.pallas_call(
        functools.partial(_layer_body, heads=heads, topk=topk,
                          exact_agg=exact_agg, imgs=imgs),
        grid=(b // imgs,),
        in_specs=[pl.BlockSpec((imgs, n, c), lambda i: (i, 0, 0))]
                 + [full(w) for w in ws],
        out_specs=pl.BlockSpec((imgs, n, co), lambda i: (i, 0, 0)),
        out_shape=jax.ShapeDtypeStruct((b, n, co), jnp.float32),
        compiler_params=pltpu.CompilerParams(
            dimension_semantics=("parallel",)),
    )(t, *ws)


def _fc_body(f_ref, w1_ref, b1_ref, w2_ref, b2_ref, o_ref, acc_ref, *, nk):
    ki = pl.program_id(0)

    @pl.when(ki == 0)
    def _init():
        acc_ref[...] = jnp.zeros_like(acc_ref)

    fb = f_ref[...].astype(jnp.bfloat16)
    wb = w1_ref[...].astype(jnp.bfloat16)
    acc_ref[...] += jnp.dot(fb, wb, preferred_element_type=jnp.float32)

    @pl.when(ki == nk - 1)
    def _fin():
        h = jnp.maximum(acc_ref[...] + b1_ref[...], 0.0)
        o_ref[...] = jnp.dot(h.astype(jnp.bfloat16),
                             w2_ref[...].astype(jnp.bfloat16),
                             preferred_element_type=jnp.float32) + b2_ref[...]


def _classifier(f, w1, b1, w2, b2, *, kblk=4096):
    b, ktot = f.shape
    hid = w1.shape[1]
    ncls = w2.shape[1]
    nk = ktot // kblk
    b1 = b1.reshape(1, -1)
    b2 = b2.reshape(1, -1)
    return pl.pallas_call(
        functools.partial(_fc_body, nk=nk),
        grid=(nk,),
        in_specs=[
            pl.BlockSpec((b, kblk), lambda i: (0, i)),
            pl.BlockSpec((kblk, hid), lambda i: (i, 0)),
            pl.BlockSpec((1, hid), lambda i: (0, 0)),
            pl.BlockSpec((hid, ncls), lambda i: (0, 0)),
            pl.BlockSpec((1, ncls), lambda i: (0, 0)),
        ],
        out_specs=pl.BlockSpec((b, ncls), lambda i: (0, 0)),
        out_shape=jax.ShapeDtypeStruct((b, ncls), jnp.float32),
        scratch_shapes=[pltpu.VMEM((b, hid), jnp.float32)],
        compiler_params=pltpu.CompilerParams(
            dimension_semantics=("arbitrary",)),
    )(f, w1, b1, w2, b2)


def _unshuffle_tokens(x, r):
    b, c, hh, ww = x.shape
    x = x.reshape(b, c, hh // r, r, ww // r, r)
    x = x.transpose(0, 1, 3, 5, 2, 4)
    x = x.reshape(b, c * r * r, (hh // r) * (ww // r))
    return x.transpose(0, 2, 1)


def kernel(x, Wc1, bc1, Wq1, Wk1, Wv1, Wo1, bo1, Wc2, bc2, Wq2, Wk2, Wv2, Wo2,
           bo2, W1, b1, W2, b2):
    t1 = _unshuffle_tokens(x, 2)
    h1 = _attn_layer(t1, [Wc1, bc1, Wq1, Wk1, Wv1, Wo1, bo1],
                     heads=4, topk=9, exact_agg=True)
    h2 = _attn_layer(h1, [Wc2, bc2, Wq2, Wk2, Wv2, Wo2, bo2],
                     heads=4, topk=9, exact_agg=False)
    b, n, co = h2.shape
    hs = int(math.isqrt(n))
    g = h2.transpose(0, 2, 1).reshape(b, co, hs, hs)
    r = 2
    g = g.reshape(b, co // (r * r), r, r, hs, hs)
    g = g.transpose(0, 1, 4, 2, 5, 3).reshape(b, co // (r * r), hs * r, hs * r)
    f = g.reshape(b, -1)
    return _classifier(f, W1, b1, W2, b2)


# speedup vs baseline: 1.0932x; 1.0526x over previous
import functools
import math

import jax
import jax.numpy as jnp
from jax.experimental import pallas as pl
from jax.experimental.pallas import tpu as pltpu

_NEG = -3.0e38


def _attn_block(t, wc, bc, wq, wk, wv, wo, bo, *, heads, topk, exact_agg):
    bf = lambda a: a.astype(jnp.bfloat16)
    dot = lambda a, b: jnp.dot(bf(a), bf(b), preferred_element_type=jnp.float32)
    conv = dot(t, wc) + bc
    q = dot(t, wq)
    k = dot(t, wk)
    v = dot(t, wv)
    n, d = q.shape
    dh = d // heads
    scale = jnp.sqrt(jnp.float32(dh))
    col = jax.lax.broadcasted_iota(jnp.int32, (n, d), 1)

    kb = bf(k)
    sims = []
    for h in range(heads):
        qm = jnp.where((col >= h * dh) & (col < (h + 1) * dh), q, 0.0)
        sims.append(jax.lax.dot_general(
            bf(qm), kb, (((1,), (1,)), ((), ())),
            preferred_element_type=jnp.float32) / scale)
    sim = jnp.concatenate(sims, axis=0)

    r = heads * n
    lanef = jax.lax.broadcasted_iota(jnp.int32, (r, n), 1).astype(jnp.float32)
    cur = sim
    rmax = None
    for i in range(topk):
        m = jnp.max(cur, axis=-1, keepdims=True)
        if i == 0:
            rmax = m
        cand = jnp.where(cur == m, lanef, 1e9)
        imin = jnp.min(cand, axis=-1, keepdims=True)
        cur = jnp.where(cand == imin, _NEG, cur)
    e = jnp.where(cur < -1.0e37, jnp.exp(sim - rmax), 0.0)
    attn = e * (1.0 / jnp.sum(e, axis=-1, keepdims=True))

    agg = jnp.zeros((n, d), dtype=jnp.float32)
    for h in range(heads):
        vm = jnp.where((col >= h * dh) & (col < (h + 1) * dh), v, 0.0)
        ah = attn[h * n:(h + 1) * n]
        if exact_agg:
            ah_hi = bf(ah)
            ah_lo = bf(ah - ah_hi.astype(jnp.float32))
            vm_hi = bf(vm)
            vm_lo = bf(vm - vm_hi.astype(jnp.float32))
            agg = (agg
                   + jnp.dot(ah_hi, vm_hi, preferred_element_type=jnp.float32)
                   + jnp.dot(ah_hi, vm_lo, preferred_element_type=jnp.float32)
                   + jnp.dot(ah_lo, vm_hi, preferred_element_type=jnp.float32))
        else:
            agg = agg + dot(ah, vm)

    cat = jnp.concatenate([conv, agg], axis=-1)
    out = dot(cat, wo) + bo
    return jnp.maximum(out, 0.0)


def _layer_body(t_ref, wc_ref, bc_ref, wq_ref, wk_ref, wv_ref, wo_ref, bo_ref,
                o_ref, *, heads, topk, exact_agg, imgs):
    for j in range(imgs):
        o_ref[j] = _attn_block(
            t_ref[j], wc_ref[...], bc_ref[...], wq_ref[...], wk_ref[...],
            wv_ref[...], wo_ref[...], bo_ref[...],
            heads=heads, topk=topk, exact_agg=exact_agg)


def _attn_layer(t, ws, *, heads, topk, exact_agg, imgs=1):
    b, n, c = t.shape
    co = ws[-2].shape[1]
    ws = [w.reshape(1, -1) if w.ndim == 1 else w for w in ws]
    full = lambda w: pl.BlockSpec(w.shape, lambda i: (0,) * w.ndim)
    return pl.pallas_call(
        functools.partial(_layer_body, heads=heads, topk=topk,
                          exact_agg=exact_agg, imgs=imgs),
        grid=(b // imgs,),
        in_specs=[pl.BlockSpec((imgs, n, c), lambda i: (i, 0, 0))]
                 + [full(w) for w in ws],
        out_specs=pl.BlockSpec((imgs, n, co), lambda i: (i, 0, 0)),
        out_shape=jax.ShapeDtypeStruct((b, n, co), jnp.float32),
        compiler_params=pltpu.CompilerParams(
            dimension_semantics=("parallel",)),
    )(t, *ws)


def _fc_body(f_ref, w1_ref, b1_ref, w2_ref, b2_ref, o_ref, acc_ref, *, nk):
    ki = pl.program_id(0)

    @pl.when(ki == 0)
    def _init():
        acc_ref[...] = jnp.zeros_like(acc_ref)

    fb = f_ref[...].astype(jnp.bfloat16)
    wb = w1_ref[...].astype(jnp.bfloat16)
    acc_ref[...] += jnp.dot(fb, wb, preferred_element_type=jnp.float32)

    @pl.when(ki == nk - 1)
    def _fin():
        h = jnp.maximum(acc_ref[...] + b1_ref[...], 0.0)
        o_ref[...] = jnp.dot(h.astype(jnp.bfloat16),
                             w2_ref[...].astype(jnp.bfloat16),
                             preferred_element_type=jnp.float32) + b2_ref[...]


def _classifier(f, w1, b1, w2, b2, *, kblk=4096):
    b, ktot = f.shape
    hid = w1.shape[1]
    ncls = w2.shape[1]
    nk = ktot // kblk
    b1 = b1.reshape(1, -1)
    b2 = b2.reshape(1, -1)
    return pl.pallas_call(
        functools.partial(_fc_body, nk=nk),
        grid=(nk,),
        in_specs=[
            pl.BlockSpec((b, kblk), lambda i: (0, i)),
            pl.BlockSpec((kblk, hid), lambda i: (i, 0)),
            pl.BlockSpec((1, hid), lambda i: (0, 0)),
            pl.BlockSpec((hid, ncls), lambda i: (0, 0)),
            pl.BlockSpec((1, ncls), lambda i: (0, 0)),
        ],
        out_specs=pl.BlockSpec((b, ncls), lambda i: (0, 0)),
        out_shape=jax.ShapeDtypeStruct((b, ncls), jnp.float32),
        scratch_shapes=[pltpu.VMEM((b, hid), jnp.float32)],
        compiler_params=pltpu.CompilerParams(
            dimension_semantics=("arbitrary",)),
    )(f, w1, b1, w2, b2)


def _unshuffle_tokens(x, r):
    b, c, hh, ww = x.shape
    x = x.reshape(b, c, hh // r, r, ww // r, r)
    x = x.transpose(0, 1, 3, 5, 2, 4)
    x = x.reshape(b, c * r * r, (hh // r) * (ww // r))
    return x.transpose(0, 2, 1)


def kernel(x, Wc1, bc1, Wq1, Wk1, Wv1, Wo1, bo1, Wc2, bc2, Wq2, Wk2, Wv2, Wo2,
           bo2, W1, b1, W2, b2):
    t1 = _unshuffle_tokens(x, 2)
    h1 = _attn_layer(t1, [Wc1, bc1, Wq1, Wk1, Wv1, Wo1, bo1],
                     heads=4, topk=9, exact_agg=True)
    h2 = _attn_layer(h1, [Wc2, bc2, Wq2, Wk2, Wv2, Wo2, bo2],
                     heads=4, topk=9, exact_agg=False)
    b, n, co = h2.shape
    hs = int(math.isqrt(n))
    g = h2.transpose(0, 2, 1).reshape(b, co, hs, hs)
    r = 2
    g = g.reshape(b, co // (r * r), r, r, hs, hs)
    g = g.transpose(0, 1, 4, 2, 5, 3).reshape(b, co // (r * r), hs * r, hs * r)
    f = g.reshape(b, -1)
    return _classifier(f, W1, b1, W2, b2)


# speedup vs baseline: 1.0964x; 1.0029x over previous
import functools
import math

import jax
import jax.numpy as jnp
from jax.experimental import pallas as pl
from jax.experimental.pallas import tpu as pltpu

_NEG = -3.0e38


def _attn_block(t, wc, bc, wq, wk, wv, wo, bo, *, heads, topk, exact_agg):
    bf = lambda a: a.astype(jnp.bfloat16)
    dot = lambda a, b: jnp.dot(bf(a), bf(b), preferred_element_type=jnp.float32)
    conv = dot(t, wc) + bc
    q = dot(t, wq)
    k = dot(t, wk)
    v = dot(t, wv)
    n, d = q.shape
    dh = d // heads
    scale = jnp.sqrt(jnp.float32(dh))
    col = jax.lax.broadcasted_iota(jnp.int32, (n, d), 1)

    kb = bf(k)
    sims = []
    for h in range(heads):
        qm = jnp.where((col >= h * dh) & (col < (h + 1) * dh), q, 0.0)
        sims.append(jax.lax.dot_general(
            bf(qm), kb, (((1,), (1,)), ((), ())),
            preferred_element_type=jnp.float32) / scale)
    sim = jnp.concatenate(sims, axis=0)

    r = heads * n
    lanef = jax.lax.broadcasted_iota(jnp.int32, (r, n), 1).astype(jnp.float32)
    cur = sim
    rmax = None
    for i in range(topk):
        m = jnp.max(cur, axis=-1, keepdims=True)
        if i == 0:
            rmax = m
        cand = jnp.where(cur == m, lanef, 1e9)
        imin = jnp.min(cand, axis=-1, keepdims=True)
        cur = jnp.where(cand == imin, _NEG, cur)
    e = jnp.where(cur < -1.0e37, jnp.exp(sim - rmax), 0.0)
    attn = e * (1.0 / jnp.sum(e, axis=-1, keepdims=True))

    agg = jnp.zeros((n, d), dtype=jnp.float32)
    for h in range(heads):
        vm = jnp.where((col >= h * dh) & (col < (h + 1) * dh), v, 0.0)
        ah = attn[h * n:(h + 1) * n]
        if exact_agg:
            ah_hi = bf(ah)
            ah_lo = bf(ah - ah_hi.astype(jnp.float32))
            vm_hi = bf(vm)
            vm_lo = bf(vm - vm_hi.astype(jnp.float32))
            agg = (agg
                   + jnp.dot(ah_hi, vm_hi, preferred_element_type=jnp.float32)
                   + jnp.dot(ah_hi, vm_lo, preferred_element_type=jnp.float32)
                   + jnp.dot(ah_lo, vm_hi, preferred_element_type=jnp.float32))
        else:
            agg = agg + dot(ah, vm)

    cat = jnp.concatenate([conv, agg], axis=-1)
    out = dot(cat, wo) + bo
    return jnp.maximum(out, 0.0)


def _layer_body(t_ref, wc_ref, bc_ref, wq_ref, wk_ref, wv_ref, wo_ref, bo_ref,
                o_ref, *, heads, topk, exact_agg, imgs):
    for j in range(imgs):
        o_ref[j] = _attn_block(
            t_ref[j], wc_ref[...], bc_ref[...], wq_ref[...], wk_ref[...],
            wv_ref[...], wo_ref[...], bo_ref[...],
            heads=heads, topk=topk, exact_agg=exact_agg)


def _attn_layer(t, ws, *, heads, topk, exact_agg, imgs=1):
    b, n, c = t.shape
    co = ws[-2].shape[1]
    ws = [w.reshape(1, -1) if w.ndim == 1 else w for w in ws]
    full = lambda w: pl.BlockSpec(w.shape, lambda i: (0,) * w.ndim)
    return pl.pallas_call(
        functools.partial(_layer_body, heads=heads, topk=topk,
                          exact_agg=exact_agg, imgs=imgs),
        grid=(b // imgs,),
        in_specs=[pl.BlockSpec((imgs, n, c), lambda i: (i, 0, 0))]
                 + [full(w) for w in ws],
        out_specs=pl.BlockSpec((imgs, n, co), lambda i: (i, 0, 0)),
        out_shape=jax.ShapeDtypeStruct((b, n, co), jnp.float32),
        compiler_params=pltpu.CompilerParams(
            dimension_semantics=("parallel",)),
    )(t, *ws)


def _fc_body(f_ref, w1_ref, b1_ref, w2_ref, b2_ref, o_ref, acc_ref, *, nk):
    ki = pl.program_id(0)

    @pl.when(ki == 0)
    def _init():
        acc_ref[...] = jnp.zeros_like(acc_ref)

    fb = f_ref[...].astype(jnp.bfloat16)
    wb = w1_ref[...].astype(jnp.bfloat16)
    acc_ref[...] += jnp.dot(fb, wb, preferred_element_type=jnp.float32)

    @pl.when(ki == nk - 1)
    def _fin():
        h = jnp.maximum(acc_ref[...] + b1_ref[...], 0.0)
        o_ref[...] = jnp.dot(h.astype(jnp.bfloat16),
                             w2_ref[...].astype(jnp.bfloat16),
                             preferred_element_type=jnp.float32) + b2_ref[...]


def _classifier(f, w1, b1, w2, b2, *, kblk=2048):
    b, ktot = f.shape
    hid = w1.shape[1]
    ncls = w2.shape[1]
    nk = ktot // kblk
    b1 = b1.reshape(1, -1)
    b2 = b2.reshape(1, -1)
    return pl.pallas_call(
        functools.partial(_fc_body, nk=nk),
        grid=(nk,),
        in_specs=[
            pl.BlockSpec((b, kblk), lambda i: (0, i)),
            pl.BlockSpec((kblk, hid), lambda i: (i, 0)),
            pl.BlockSpec((1, hid), lambda i: (0, 0)),
            pl.BlockSpec((hid, ncls), lambda i: (0, 0)),
            pl.BlockSpec((1, ncls), lambda i: (0, 0)),
        ],
        out_specs=pl.BlockSpec((b, ncls), lambda i: (0, 0)),
        out_shape=jax.ShapeDtypeStruct((b, ncls), jnp.float32),
        scratch_shapes=[pltpu.VMEM((b, hid), jnp.float32)],
        compiler_params=pltpu.CompilerParams(
            dimension_semantics=("arbitrary",)),
    )(f, w1, b1, w2, b2)


def _unshuffle_tokens(x, r):
    b, c, hh, ww = x.shape
    x = x.reshape(b, c, hh // r, r, ww // r, r)
    x = x.transpose(0, 1, 3, 5, 2, 4)
    x = x.reshape(b, c * r * r, (hh // r) * (ww // r))
    return x.transpose(0, 2, 1)


def kernel(x, Wc1, bc1, Wq1, Wk1, Wv1, Wo1, bo1, Wc2, bc2, Wq2, Wk2, Wv2, Wo2,
           bo2, W1, b1, W2, b2):
    t1 = _unshuffle_tokens(x, 2)
    h1 = _attn_layer(t1, [Wc1, bc1, Wq1, Wk1, Wv1, Wo1, bo1],
                     heads=4, topk=9, exact_agg=True)
    h2 = _attn_layer(h1, [Wc2, bc2, Wq2, Wk2, Wv2, Wo2, bo2],
                     heads=4, topk=9, exact_agg=False)
    b, n, co = h2.shape
    hs = int(math.isqrt(n))
    g = h2.transpose(0, 2, 1).reshape(b, co, hs, hs)
    r = 2
    g = g.reshape(b, co // (r * r), r, r, hs, hs)
    g = g.transpose(0, 1, 4, 2, 5, 3).reshape(b, co // (r * r), hs * r, hs * r)
    f = g.reshape(b, -1)
    return _classifier(f, W1, b1, W2, b2)
